# Initial kernel scaffold; baseline (speedup 1.0000x reference)
import jax
import jax.numpy as jnp
from jax.experimental import pallas as pl


def kernel(input_bids, input_wids, bottom_emb, word_emb, gamma, beta):
    raise NotImplementedError("write your pallas kernel here")


# speedup vs baseline: 2.9969x; 2.9969x over previous
import functools

import jax
import jax.numpy as jnp
from jax import lax
from jax.experimental import pallas as pl
from jax.experimental.pallas import tpu as pltpu
from jax.experimental.pallas import tpu_sc as plsc

VOCAB = 1000000
STYLE = 18
WPAD = 24
B = 16384
L = 200
EPS = 1e-12

N = B * L
NW = 32
TOK_PER_W = N // NW
CHUNK = 1024
GATHER = 128
N_GATHER = CHUNK // GATHER
GROUPS = CHUNK // 16
N_CHUNK = TOK_PER_W // CHUNK


def _rsqrt(v):
    i = plsc.bitcast(v, jnp.int32)
    i = jnp.int32(0x5F3759DF) - (i >> 1)
    y = plsc.bitcast(i, jnp.float32)
    for _ in range(3):
        y = y * (1.5 - 0.5 * v * y * y)
    return y


def _body(wids_hbm, bids_hbm, bot_hbm, word_hbm, gam_hbm, bet_hbm, out_hbm,
          wid_v, bid_v, rows_v, out_v, bot_v, gam_v, bet_v, sem):
    nc = 2
    w = lax.axis_index("s") * nc + lax.axis_index("c")
    base_w = w * TOK_PER_W

    pltpu.sync_copy(bot_hbm, bot_v)
    pltpu.sync_copy(gam_hbm, gam_v)
    pltpu.sync_copy(bet_hbm, bet_v)
    g0, g1 = gam_v[pl.ds(0, 16)], gam_v[pl.ds(16, 16)]
    b0, b1 = bet_v[pl.ds(0, 16)], bet_v[pl.ds(16, 16)]
    gam = [g0[d] for d in range(16)] + [g1[0], g1[1]]
    bet = [b0[d] for d in range(16)] + [b1[0], b1[1]]

    lanes = lax.broadcasted_iota(jnp.int32, (16,), 0)

    def chunk_body(ci, _):
        tok = base_w + ci * CHUNK
        pltpu.sync_copy(wids_hbm.at[pl.ds(tok // GATHER, N_GATHER), :],
                        wid_v)
        pltpu.sync_copy(bids_hbm.at[pl.ds(tok, CHUNK)], bid_v)
        cps = []
        for j in range(N_GATHER):
            sl = pl.ds(j * GATHER, GATHER)
            cps.append(pltpu.async_copy(
                word_hbm.at[wid_v.at[j]], rows_v.at[sl], sem))
        for c in cps:
            c.wait()

        def group_body(g, _):
            ridx = lanes + g * 16
            bidv = bid_v[pl.ds(g * 16, 16)] * STYLE
            oidx = ridx * STYLE
            x = []
            for d in range(STYLE):
                cd = jnp.full((16,), d, jnp.int32)
                wv = plsc.load_gather(rows_v, [ridx, cd])
                bv = plsc.load_gather(bot_v, [bidv + d])
                x.append(wv * bv)
            s = x[0]
            for d in range(1, STYLE):
                s = s + x[d]
            m = s * (1.0 / STYLE)
            t = [xd - m for xd in x]
            q = t[0] * t[0]
            for d in range(1, STYLE):
                q = q + t[d] * t[d]
            r = _rsqrt(q * (1.0 / STYLE) + EPS)
            for d in range(STYLE):
                yd = t[d] * (r * gam[d]) + bet[d]
                plsc.store_scatter(out_v, [oidx + d], yd)
            return None

        lax.fori_loop(0, GROUPS, group_body, None)
        pltpu.sync_copy(out_v, out_hbm.at[pl.ds(tok * STYLE, CHUNK * STYLE)])
        return None

    lax.fori_loop(0, N_CHUNK, chunk_body, None)


@jax.jit
def _run(wids, bids, bottom_flat, word_pad, gamma32, beta32):
    mesh = plsc.VectorSubcoreMesh(core_axis_name="c", subcore_axis_name="s")
    f = functools.partial(
        pl.kernel,
        mesh=mesh,
        out_type=jax.ShapeDtypeStruct((N * STYLE,), jnp.float32),
        scratch_types=[
            pltpu.VMEM((N_GATHER, GATHER), jnp.int32),
            pltpu.VMEM((CHUNK,), jnp.int32),
            pltpu.VMEM((CHUNK, WPAD), jnp.float32),
            pltpu.VMEM((CHUNK * STYLE,), jnp.float32),
            pltpu.VMEM((STYLE * STYLE,), jnp.float32),
            pltpu.VMEM((32,), jnp.float32),
            pltpu.VMEM((32,), jnp.float32),
            pltpu.SemaphoreType.DMA,
        ],
        compiler_params=pltpu.CompilerParams(
            needs_layout_passes=False, use_tc_tiling_on_sc=False),
    )(_body)
    return f(wids, bids, bottom_flat, word_pad, gamma32, beta32)


def kernel(input_bids, input_wids, bottom_emb, word_emb, gamma, beta):
    wids = input_wids.reshape(-1, GATHER).astype(jnp.int32)
    bids = input_bids.reshape(-1).astype(jnp.int32)
    word_pad = jnp.pad(word_emb, ((0, 0), (0, WPAD - STYLE)))
    bottom_flat = bottom_emb.reshape(-1)
    gam32 = jnp.zeros((32,), jnp.float32).at[:STYLE].set(gamma)
    bet32 = jnp.zeros((32,), jnp.float32).at[:STYLE].set(beta)
    out = _run(wids, bids, bottom_flat, word_pad, gam32, bet32)
    return out.reshape(B, L, STYLE)
